# baseline (speedup 1.0000x reference)
import functools

import jax
import jax.numpy as jnp
from jax import lax
from jax.experimental import pallas as pl
from jax.experimental.pallas import tpu as pltpu
from jax.experimental.pallas import tpu_sc as plsc

EMB = 32
BATCH = 16384
NROWS = 1000000

NC = 2
NS = 16
L = 16
NW = NC * NS
BPW = BATCH // NW
KV = BPW // L


def _body(user_hbm, item_hbm, uid_hbm, iid_hbm, out_hbm, *scr):
    uidx = scr[0:EMB]
    iidx = scr[EMB:2 * EMB]
    ubuf = scr[2 * EMB:3 * EMB]
    ibuf = scr[3 * EMB:4 * EMB]
    outv_v = scr[4 * EMB]
    sem_u = scr[4 * EMB + 1]
    sem_i = scr[4 * EMB + 2]

    wid = lax.axis_index("s") * NC + lax.axis_index("c")
    base = wid * BPW

    pltpu.sync_copy(uid_hbm.at[pl.ds(base, BPW)], uidx[0])
    pltpu.sync_copy(iid_hbm.at[pl.ds(base, BPW)], iidx[0])

    def mkidx(k, carry):
        s = pl.ds(k * L, L)
        u0 = uidx[0][s]
        i0 = iidx[0][s]
        for e in range(1, EMB):
            uidx[e][s] = u0 + e * NROWS
            iidx[e][s] = i0 + e * NROWS
        return carry

    lax.fori_loop(0, KV, mkidx, 0)

    copies = []
    for e in range(EMB):
        copies.append(pltpu.async_copy(
            user_hbm.at[uidx[e]], ubuf[e], sem_u))
        copies.append(pltpu.async_copy(
            item_hbm.at[iidx[e]], ibuf[e], sem_i))
    for c in copies:
        c.wait()

    def chunk(k, carry):
        s = pl.ds(k * L, L)
        acc = ubuf[0][s] * ibuf[0][s]
        for e in range(1, EMB):
            acc = acc + ubuf[e][s] * ibuf[e][s]
        outv_v[s] = acc
        return carry

    lax.fori_loop(0, KV, chunk, 0)
    pltpu.sync_copy(outv_v, out_hbm.at[pl.ds(base, BPW)])


@jax.jit
def kernel(user_table, item_table, u_id, i_id):
    mesh = plsc.VectorSubcoreMesh(core_axis_name="c", subcore_axis_name="s",
                                  num_cores=NC, num_subcores=NS)
    k = functools.partial(
        pl.kernel,
        out_type=jax.ShapeDtypeStruct((BATCH,), jnp.float32),
        mesh=mesh,
        scratch_types=(
            [pltpu.VMEM((BPW,), jnp.int32) for _ in range(2 * EMB)]
            + [pltpu.VMEM((BPW,), jnp.float32) for _ in range(2 * EMB)]
            + [pltpu.VMEM((BPW,), jnp.float32),
               pltpu.SemaphoreType.DMA,
               pltpu.SemaphoreType.DMA]
        ),
    )(_body)
    return k(user_table.T.reshape(-1), item_table.T.reshape(-1),
             u_id.astype(jnp.int32), i_id.astype(jnp.int32))
